# Initial kernel scaffold; baseline (speedup 1.0000x reference)
import jax
import jax.numpy as jnp
from jax.experimental import pallas as pl


def kernel(x, features):
    raise NotImplementedError("write your pallas kernel here")


# speedup vs baseline: 2.0686x; 2.0686x over previous
import functools

import jax
import jax.numpy as jnp
from jax import lax
from jax.experimental import pallas as pl
from jax.experimental.pallas import tpu as pltpu
from jax.experimental.pallas import tpu_sc as plsc

H = 1024
W = 1024
C = 32
N = 524288

_INFO = plsc.get_sparse_core_info()
NC = _INFO.num_cores
NS = _INFO.num_subcores
NW = NC * NS
PTS = N // NW
P = 128
NCHUNK = PTS // P
NG = P // 16

_mesh = plsc.VectorSubcoreMesh(core_axis_name="c", subcore_axis_name="s")


@functools.partial(
    pl.kernel,
    mesh=_mesh,
    out_type=jax.ShapeDtypeStruct((N, C), jnp.float32),
    scratch_types=[
        pltpu.VMEM((P,), jnp.float32),
        pltpu.VMEM((P,), jnp.float32),
        pltpu.VMEM((P,), jnp.float32),
        pltpu.VMEM((P,), jnp.float32),
        pltpu.VMEM((P,), jnp.int32),
        pltpu.VMEM((P,), jnp.int32),
        pltpu.VMEM((P,), jnp.int32),
        pltpu.VMEM((P,), jnp.int32),
        pltpu.VMEM((P, C), jnp.float32),
        pltpu.VMEM((P, C), jnp.float32),
        pltpu.VMEM((P, C), jnp.float32),
        pltpu.VMEM((P, C), jnp.float32),
        pltpu.VMEM((P, C), jnp.float32),
        pltpu.SemaphoreType.DMA,
    ],
    compiler_params=pltpu.CompilerParams(use_tc_tiling_on_sc=False),
)
def _grid_sample(cx_hbm, cy_hbm, tab_hbm, out_hbm,
                 cx_v, cy_v, wx_v, wy_v,
                 i0_v, i1_v, i2_v, i3_v,
                 r0_v, r1_v, r2_v, r3_v,
                 ob_v, sem):
    wid = lax.axis_index("s") * NC + lax.axis_index("c")
    tile_base = wid * PTS

    def idx_group(g, carry):
        s = g * 16
        sl = pl.ds(s, 16)
        lx = (cx_v[sl] + 0.5) * (W - 1.0)
        ly = (cy_v[sl] + 0.5) * (H - 1.0)
        x0 = lx.astype(jnp.int32)
        y0 = ly.astype(jnp.int32)
        wx_v[sl] = lx - x0.astype(jnp.float32)
        wy_v[sl] = ly - y0.astype(jnp.float32)
        x1 = jnp.minimum(x0 + 1, W - 1)
        y0w = y0 * W
        y1w = jnp.minimum(y0 + 1, H - 1) * W
        i0_v[sl] = y0w + x0
        i1_v[sl] = y0w + x1
        i2_v[sl] = y1w + x0
        i3_v[sl] = y1w + x1
        return carry

    def interp_group(g, carry):
        s = g * 16
        wxv = wx_v[pl.ds(s, 16)]
        wyv = wy_v[pl.ds(s, 16)]
        for j in range(16):
            p = s + j
            wxj = jnp.full((16,), wxv[j])
            wyj = jnp.full((16,), wyv[j])
            for h in range(C // 16):
                csl = pl.ds(h * 16, 16)
                g00 = r0_v[p, csl]
                g01 = r1_v[p, csl]
                g10 = r2_v[p, csl]
                g11 = r3_v[p, csl]
                top = g00 + wxj * (g01 - g00)
                bot = g10 + wxj * (g11 - g10)
                ob_v[p, csl] = top + wyj * (bot - top)
        return carry

    def chunk_body(ci, carry):
        base = tile_base + ci * P
        pltpu.sync_copy(cx_hbm.at[pl.ds(base, P)], cx_v)
        pltpu.sync_copy(cy_hbm.at[pl.ds(base, P)], cy_v)
        lax.fori_loop(0, NG, idx_group, 0, unroll=True)
        cps = [
            pltpu.async_copy(tab_hbm.at[i0_v], r0_v, sem),
            pltpu.async_copy(tab_hbm.at[i1_v], r1_v, sem),
            pltpu.async_copy(tab_hbm.at[i2_v], r2_v, sem),
            pltpu.async_copy(tab_hbm.at[i3_v], r3_v, sem),
        ]
        for cp in cps:
            cp.wait()
        lax.fori_loop(0, NG, interp_group, 0)
        pltpu.sync_copy(ob_v, out_hbm.at[pl.ds(base, P)])
        return carry

    lax.fori_loop(0, NCHUNK, chunk_body, 0)


def kernel(x, features):
    xt = x.reshape(N, 2).T
    tab = features.reshape(H * W, C)
    out = _grid_sample(xt[0], xt[1], tab)
    return out.reshape(1, N, C)


# speedup vs baseline: 2.4774x; 1.1976x over previous
import functools

import jax
import jax.numpy as jnp
from jax import lax
from jax.experimental import pallas as pl
from jax.experimental.pallas import tpu as pltpu
from jax.experimental.pallas import tpu_sc as plsc

H = 1024
W = 1024
C = 32
N = 524288

_INFO = plsc.get_sparse_core_info()
NC = _INFO.num_cores
NS = _INFO.num_subcores
NW = NC * NS
PTS = N // NW
P = 128
NCHUNK = PTS // P
NG = P // 16
NBUF = 4

_mesh = plsc.VectorSubcoreMesh(core_axis_name="c", subcore_axis_name="s")


@functools.partial(
    pl.kernel,
    mesh=_mesh,
    out_type=jax.ShapeDtypeStruct((N, C), jnp.float32),
    scratch_types=[
        pltpu.VMEM((PTS,), jnp.float32),
        pltpu.VMEM((PTS,), jnp.float32),
        pltpu.VMEM((NBUF, P), jnp.float32),
        pltpu.VMEM((NBUF, P), jnp.float32),
        pltpu.VMEM((NBUF * 4, P), jnp.int32),
        pltpu.VMEM((NBUF * 4, P, C), jnp.float32),
        pltpu.VMEM((NBUF, P, C), jnp.float32),
        [pltpu.SemaphoreType.DMA] * NBUF,
        pltpu.SemaphoreType.DMA,
    ],
    compiler_params=pltpu.CompilerParams(use_tc_tiling_on_sc=False),
)
def _grid_sample(cx_hbm, cy_hbm, tab_hbm, out_hbm,
                 cx_v, cy_v, wx_v, wy_v, i_v, r_v, ob_v, gsems, ssem):
    wid = lax.axis_index("s") * NC + lax.axis_index("c")
    tile_base = wid * PTS

    def idx_and_fire(c, b):
        for g in range(NG):
            s = c * P + g * 16
            sl = pl.ds(s, 16)
            lx = (cx_v[sl] + 0.5) * (W - 1.0)
            ly = (cy_v[sl] + 0.5) * (H - 1.0)
            x0 = lx.astype(jnp.int32)
            y0 = ly.astype(jnp.int32)
            gs = pl.ds(g * 16, 16)
            wx_v[b, gs] = lx - x0.astype(jnp.float32)
            wy_v[b, gs] = ly - y0.astype(jnp.float32)
            x1 = jnp.minimum(x0 + 1, W - 1)
            y0w = y0 * W
            y1w = jnp.minimum(y0 + 1, H - 1) * W
            i_v[4 * b + 0, gs] = y0w + x0
            i_v[4 * b + 1, gs] = y0w + x1
            i_v[4 * b + 2, gs] = y1w + x0
            i_v[4 * b + 3, gs] = y1w + x1
        for k in range(4):
            pltpu.async_copy(
                tab_hbm.at[i_v.at[4 * b + k]], r_v.at[4 * b + k], gsems[b])

    def wait_gathers(b):
        for k in range(4):
            pltpu.make_async_copy(
                tab_hbm.at[i_v.at[4 * b + k]], r_v.at[4 * b + k],
                gsems[b]).wait()

    def interp(b):
        def group(g, carry):
            s = g * 16
            wxv = wx_v[b, pl.ds(s, 16)]
            wyv = wy_v[b, pl.ds(s, 16)]
            for j in range(16):
                p = s + j
                wxj = jnp.full((16,), wxv[j])
                wyj = jnp.full((16,), wyv[j])
                for h in range(C // 16):
                    csl = pl.ds(h * 16, 16)
                    g00 = r_v[4 * b + 0, p, csl]
                    g01 = r_v[4 * b + 1, p, csl]
                    g10 = r_v[4 * b + 2, p, csl]
                    g11 = r_v[4 * b + 3, p, csl]
                    top = g00 + wxj * (g01 - g00)
                    bot = g10 + wxj * (g11 - g10)
                    ob_v[b, p, csl] = top + wyj * (bot - top)
            return carry
        lax.fori_loop(0, NG, group, 0)

    def fire_store(c, b):
        pltpu.async_copy(
            ob_v.at[b], out_hbm.at[pl.ds(tile_base + c * P, P)], ssem)

    def drain_store():
        pltpu.make_async_copy(
            ob_v.at[0], out_hbm.at[pl.ds(tile_base, P)], ssem).wait()

    pltpu.sync_copy(cx_hbm.at[pl.ds(tile_base, PTS)], cx_v)
    pltpu.sync_copy(cy_hbm.at[pl.ds(tile_base, PTS)], cy_v)

    for b in range(NBUF - 1):
        idx_and_fire(b, b)

    @pl.loop(0, NCHUNK, step=NBUF)
    def _outer(i):
        for b in range(NBUF):
            c = i + b

            @pl.when(c + NBUF - 1 < NCHUNK)
            def _():
                idx_and_fire(c + NBUF - 1, (b + NBUF - 1) % NBUF)

            @pl.when(c >= NBUF)
            def _():
                drain_store()

            wait_gathers(b)
            interp(b)
            fire_store(c, b)

    for _ in range(NBUF):
        drain_store()


def kernel(x, features):
    xt = x.reshape(N, 2).T
    tab = features.reshape(H * W, C)
    out = _grid_sample(xt[0], xt[1], tab)
    return out.reshape(1, N, C)
